# baseline (speedup 1.0000x reference)
import functools

import jax
import jax.numpy as jnp
from jax import lax
from jax.experimental import pallas as pl
from jax.experimental.pallas import tpu as pltpu
from jax.experimental.pallas import tpu_sc as plsc

VOCAB = 1000000
DIM = 32
PAD = 128
BATCH = 4096
HIST = 200

NC = 2
NS = 16
NW = NC * NS


BLK = 512
NFULL = VOCAB // BLK
TAIL_V0 = NFULL * BLK
TAIL_N = VOCAB - TAIL_V0

_mesh = plsc.VectorSubcoreMesh(core_axis_name="c", subcore_axis_name="s")


@functools.partial(
    pl.kernel,
    mesh=_mesh,
    out_type=jax.ShapeDtypeStruct((VOCAB * DIM,), jnp.float32),
    scratch_types=[
        pltpu.VMEM((DIM, BLK), jnp.float32),
        pltpu.VMEM((DIM, BLK), jnp.float32),
        pltpu.VMEM((DIM, BLK), jnp.float32),
        pltpu.VMEM((BLK * DIM,), jnp.float32),
        pltpu.VMEM((BLK * DIM,), jnp.float32),
        pltpu.VMEM((BLK * DIM,), jnp.float32),
        pltpu.VMEM((TAIL_N * DIM,), jnp.float32),
        pltpu.SemaphoreType.DMA,
        pltpu.SemaphoreType.DMA,
        pltpu.SemaphoreType.DMA,
        pltpu.SemaphoreType.DMA,
        pltpu.SemaphoreType.DMA,
        pltpu.SemaphoreType.DMA,
    ],
    compiler_params=pltpu.CompilerParams(
        use_tc_tiling_on_sc=True,
        needs_layout_passes=False,
        disable_bounds_checks=True,
    ),
)
def _retile(tt_hbm, tail_hbm, out_hbm, in0, in1, in2, ou0, ou1, ou2, tail_v,
            is0, is1, is2, os0, os1, os2):
    in_v = (in0, in1, in2)
    out_v = (ou0, ou1, ou2)
    wid = lax.axis_index("s") * NC + lax.axis_index("c")
    isem = (is0, is1, is2)
    osem = (os0, os1, os2)
    nblocks = jnp.where(wid < NFULL % NW, NFULL // NW + 1, NFULL // NW)

    iota16 = lax.iota(jnp.int32, 16)
    iota512 = iota16 * DIM

    def load(i, h):
        blk = wid + i * NW
        for tr in range(DIM // 8):
            pltpu.async_copy(
                tt_hbm.at[pl.ds(tr * 8, 8), pl.ds(blk * BLK, BLK)],
                in_v[h].at[pl.ds(tr * 8, 8)],
                isem[h],
            )

    def out_slice(i):
        blk = wid + i * NW
        return out_hbm.at[pl.ds(blk * (BLK * DIM), BLK * DIM)]

    def transpose(h):
        @plsc.parallel_loop(0, DIM, step=1, unroll=4)
        def _d(d):
            dsplat = jnp.full((16,), d, jnp.int32)
            for j in range(BLK // 16):
                x = plsc.load_gather(in_v[h], [dsplat, iota16 + (16 * j)])
                plsc.store_scatter(out_v[h], [iota512 + (16 * j * DIM + d)], x)

    def process(i, h, wait_out, refill):
        pltpu.make_async_copy(tt_hbm.at[:, pl.ds(0, BLK)], in_v[h], isem[h]).wait()
        if wait_out:
            pltpu.make_async_copy(out_v[h], out_slice(i), osem[h]).wait()
        transpose(h)
        pltpu.async_copy(out_v[h], out_slice(i), osem[h])
        if refill:
            @pl.when(i + 3 < nblocks)
            def _():
                load(i + 3, h)

    @pl.when(wid == 0)
    def _tail():
        pltpu.sync_copy(tail_hbm, tail_v)
        pltpu.sync_copy(tail_v, out_hbm.at[pl.ds(TAIL_V0 * DIM, TAIL_N * DIM)])

    load(0, 0)

    @pl.when(nblocks > 1)
    def _l1():
        load(1, 1)

    @pl.when(nblocks > 2)
    def _l2():
        load(2, 2)

    process(0, 0, False, True)

    @pl.when(nblocks > 1)
    def _p1():
        process(1, 1, False, True)

    @pl.when(nblocks > 2)
    def _p2():
        process(2, 2, False, True)

    @pl.loop(3, jnp.maximum(nblocks, 3), step=3)
    def _blocks(i0):
        process(i0, 0, True, True)

        @pl.when(i0 + 1 < nblocks)
        def _s1():
            process(i0 + 1, 1, True, True)

        @pl.when(i0 + 2 < nblocks)
        def _s2():
            process(i0 + 2, 2, True, True)

    pltpu.make_async_copy(out_v[0], out_slice(0), osem[0]).wait()

    @pl.when(nblocks > 1)
    def _drain1():
        pltpu.make_async_copy(out_v[1], out_slice(0), osem[1]).wait()

    @pl.when(nblocks > 2)
    def _drain2():
        pltpu.make_async_copy(out_v[2], out_slice(0), osem[2]).wait()


ROWS_PER_W = BATCH // NW
HCHUNKS = ((0, 104), (104, 96))
NGROUP = ROWS_PER_W // 4
GR = 4


@functools.partial(
    pl.kernel,
    mesh=_mesh,
    out_type=jax.ShapeDtypeStruct((BATCH, HIST, PAD), jnp.float32),
    scratch_types=[
        pltpu.VMEM((ROWS_PER_W, HIST), jnp.int32),
        pltpu.VMEM((2, GR, HIST, DIM), jnp.float32),
        pltpu.SemaphoreType.DMA,
        pltpu.SemaphoreType.DMA,
        pltpu.SemaphoreType.DMA,
        pltpu.SemaphoreType.DMA,
    ],
    compiler_params=pltpu.CompilerParams(use_tc_tiling_on_sc=False),
)
def _emb_gather(idx_hbm, table_hbm, out_hbm, idx_v, rows_v, gs0, gs1, os0, os1):
    wid = lax.axis_index("s") * NC + lax.axis_index("c")
    rbase = wid * ROWS_PER_W
    gsem = (gs0, gs1)
    osem = (os0, os1)

    pltpu.sync_copy(idx_hbm.at[pl.ds(rbase, ROWS_PER_W)], idx_v)

    def fire_gathers(g, h):
        for k in range(GR):
            for off, n in HCHUNKS:
                pltpu.async_copy(
                    table_hbm.at[idx_v.at[g * GR + k, pl.ds(off, n)]],
                    rows_v.at[h, k, pl.ds(off, n)],
                    gsem[h],
                )

    def process(g, h, refill):
        out_slice = out_hbm.at[pl.ds(rbase + g * GR, GR), :, pl.ds(0, DIM)]
        pltpu.make_async_copy(out_slice, rows_v.at[h], gsem[h]).wait()
        out_copy = pltpu.make_async_copy(rows_v.at[h], out_slice, osem[h])
        out_copy.start()
        out_copy.wait()
        if refill:
            fire_gathers(g + 2, h)

    fire_gathers(0, 0)
    fire_gathers(1, 1)

    @pl.loop(0, NGROUP - 2, step=2)
    def _groups(g0):
        process(g0, 0, True)
        process(g0 + 1, 1, True)

    process(NGROUP - 2, 0, False)
    process(NGROUP - 1, 1, False)


def kernel(token_indices, table):
    tail = table[TAIL_V0:].reshape(TAIL_N * DIM)
    tlin = _retile(table.T, tail).reshape(VOCAB, DIM)
    out = _emb_gather(token_indices.astype(jnp.int32), tlin)
    return out[:, :, :DIM]


# speedup vs baseline: 1.3407x; 1.3407x over previous
import functools

import jax
import jax.numpy as jnp
from jax import lax
from jax.experimental import pallas as pl
from jax.experimental.pallas import tpu as pltpu
from jax.experimental.pallas import tpu_sc as plsc

VOCAB = 1000000
DIM = 32
PAD = 128
BATCH = 4096
HIST = 200

NC = 2
NS = 16
NW = NC * NS


BLK = 512
NFULL = VOCAB // BLK
TAIL_V0 = NFULL * BLK
TAIL_N = VOCAB - TAIL_V0

_mesh = plsc.VectorSubcoreMesh(core_axis_name="c", subcore_axis_name="s")


@functools.partial(
    pl.kernel,
    mesh=_mesh,
    out_type=jax.ShapeDtypeStruct((VOCAB * DIM,), jnp.float32),
    scratch_types=[
        pltpu.VMEM((DIM, BLK), jnp.float32),
        pltpu.VMEM((DIM, BLK), jnp.float32),
        pltpu.VMEM((DIM, BLK), jnp.float32),
        pltpu.VMEM((BLK * DIM,), jnp.float32),
        pltpu.VMEM((BLK * DIM,), jnp.float32),
        pltpu.VMEM((BLK * DIM,), jnp.float32),
        pltpu.VMEM((TAIL_N * DIM,), jnp.float32),
        pltpu.SemaphoreType.DMA,
        pltpu.SemaphoreType.DMA,
        pltpu.SemaphoreType.DMA,
        pltpu.SemaphoreType.DMA,
        pltpu.SemaphoreType.DMA,
        pltpu.SemaphoreType.DMA,
    ],
    compiler_params=pltpu.CompilerParams(
        use_tc_tiling_on_sc=True,
        needs_layout_passes=False,
        disable_bounds_checks=True,
    ),
)
def _retile(tt_hbm, tail_hbm, out_hbm, in0, in1, in2, ou0, ou1, ou2, tail_v,
            is0, is1, is2, os0, os1, os2):
    in_v = (in0, in1, in2)
    out_v = (ou0, ou1, ou2)
    wid = lax.axis_index("s") * NC + lax.axis_index("c")
    isem = (is0, is1, is2)
    osem = (os0, os1, os2)
    nblocks = jnp.where(wid < NFULL % NW, NFULL // NW + 1, NFULL // NW)

    iota16 = lax.iota(jnp.int32, 16)
    diag_v = [(iota16 + k) % 16 for k in range(16)]
    diag_dst = [((iota16 + k) % 16) * DIM + iota16 for k in range(16)]

    def load(i, h):
        blk = wid + i * NW
        for tr in range(DIM // 8):
            pltpu.async_copy(
                tt_hbm.at[pl.ds(tr * 8, 8), pl.ds(blk * BLK, BLK)],
                in_v[h].at[pl.ds(tr * 8, 8)],
                isem[h],
            )

    def out_slice(i):
        blk = wid + i * NW
        return out_hbm.at[pl.ds(blk * (BLK * DIM), BLK * DIM)]

    def transpose(h):
        @plsc.parallel_loop(0, BLK // 16, step=1, unroll=2)
        def _m(m):
            for dh in range(DIM // 16):
                d_vec = iota16 + (16 * dh)
                for k in range(16):
                    x = plsc.load_gather(in_v[h], [d_vec, diag_v[k] + 16 * m])
                    plsc.store_scatter(
                        out_v[h], [diag_dst[k] + (DIM * 16 * m + 16 * dh)], x
                    )

    def process(i, h, wait_out, refill):
        pltpu.make_async_copy(tt_hbm.at[:, pl.ds(0, BLK)], in_v[h], isem[h]).wait()
        if wait_out:
            pltpu.make_async_copy(out_v[h], out_slice(i), osem[h]).wait()
        transpose(h)
        pltpu.async_copy(out_v[h], out_slice(i), osem[h])
        if refill:
            @pl.when(i + 3 < nblocks)
            def _():
                load(i + 3, h)

    @pl.when(wid == 0)
    def _tail():
        pltpu.sync_copy(tail_hbm, tail_v)
        pltpu.sync_copy(tail_v, out_hbm.at[pl.ds(TAIL_V0 * DIM, TAIL_N * DIM)])

    load(0, 0)

    @pl.when(nblocks > 1)
    def _l1():
        load(1, 1)

    @pl.when(nblocks > 2)
    def _l2():
        load(2, 2)

    process(0, 0, False, True)

    @pl.when(nblocks > 1)
    def _p1():
        process(1, 1, False, True)

    @pl.when(nblocks > 2)
    def _p2():
        process(2, 2, False, True)

    @pl.loop(3, jnp.maximum(nblocks, 3), step=3)
    def _blocks(i0):
        process(i0, 0, True, True)

        @pl.when(i0 + 1 < nblocks)
        def _s1():
            process(i0 + 1, 1, True, True)

        @pl.when(i0 + 2 < nblocks)
        def _s2():
            process(i0 + 2, 2, True, True)

    pltpu.make_async_copy(out_v[0], out_slice(0), osem[0]).wait()

    @pl.when(nblocks > 1)
    def _drain1():
        pltpu.make_async_copy(out_v[1], out_slice(0), osem[1]).wait()

    @pl.when(nblocks > 2)
    def _drain2():
        pltpu.make_async_copy(out_v[2], out_slice(0), osem[2]).wait()


ROWS_PER_W = BATCH // NW
HCHUNKS = ((0, 104), (104, 96))
NGROUP = ROWS_PER_W // 4
GR = 4


@functools.partial(
    pl.kernel,
    mesh=_mesh,
    out_type=jax.ShapeDtypeStruct((BATCH, HIST, PAD), jnp.float32),
    scratch_types=[
        pltpu.VMEM((ROWS_PER_W, HIST), jnp.int32),
        pltpu.VMEM((2, GR, HIST, DIM), jnp.float32),
        pltpu.SemaphoreType.DMA,
        pltpu.SemaphoreType.DMA,
        pltpu.SemaphoreType.DMA,
        pltpu.SemaphoreType.DMA,
    ],
    compiler_params=pltpu.CompilerParams(use_tc_tiling_on_sc=False),
)
def _emb_gather(idx_hbm, table_hbm, out_hbm, idx_v, rows_v, gs0, gs1, os0, os1):
    wid = lax.axis_index("s") * NC + lax.axis_index("c")
    rbase = wid * ROWS_PER_W
    gsem = (gs0, gs1)
    osem = (os0, os1)

    pltpu.sync_copy(idx_hbm.at[pl.ds(rbase, ROWS_PER_W)], idx_v)

    def fire_gathers(g, h):
        for k in range(GR):
            for off, n in HCHUNKS:
                pltpu.async_copy(
                    table_hbm.at[idx_v.at[g * GR + k, pl.ds(off, n)]],
                    rows_v.at[h, k, pl.ds(off, n)],
                    gsem[h],
                )

    def process(g, h, refill):
        out_slice = out_hbm.at[pl.ds(rbase + g * GR, GR), :, pl.ds(0, DIM)]
        pltpu.make_async_copy(out_slice, rows_v.at[h], gsem[h]).wait()
        out_copy = pltpu.make_async_copy(rows_v.at[h], out_slice, osem[h])
        out_copy.start()
        out_copy.wait()
        if refill:
            fire_gathers(g + 2, h)

    fire_gathers(0, 0)
    fire_gathers(1, 1)

    @pl.loop(0, NGROUP - 2, step=2)
    def _groups(g0):
        process(g0, 0, True)
        process(g0 + 1, 1, True)

    process(NGROUP - 2, 0, False)
    process(NGROUP - 1, 1, False)


def kernel(token_indices, table):
    tail = table[TAIL_V0:].reshape(TAIL_N * DIM)
    tlin = _retile(table.T, tail).reshape(VOCAB, DIM)
    out = _emb_gather(token_indices.astype(jnp.int32), tlin)
    return out[:, :, :DIM]


# speedup vs baseline: 1.4580x; 1.0875x over previous
import functools

import jax
import jax.numpy as jnp
from jax import lax
from jax.experimental import pallas as pl
from jax.experimental.pallas import tpu as pltpu
from jax.experimental.pallas import tpu_sc as plsc

VOCAB = 1000000
DIM = 32
PAD = 128
BATCH = 4096
HIST = 200

NC = 2
NS = 16
NW = NC * NS


BLK = 512
NFULL = VOCAB // BLK
TAIL_V0 = NFULL * BLK
TAIL_N = VOCAB - TAIL_V0

_mesh = plsc.VectorSubcoreMesh(core_axis_name="c", subcore_axis_name="s")


@functools.partial(
    pl.kernel,
    mesh=_mesh,
    out_type=jax.ShapeDtypeStruct((VOCAB * DIM,), jnp.float32),
    scratch_types=[
        pltpu.VMEM((DIM, BLK), jnp.float32),
        pltpu.VMEM((DIM, BLK), jnp.float32),
        pltpu.VMEM((DIM, BLK), jnp.float32),
        pltpu.VMEM((BLK * DIM,), jnp.float32),
        pltpu.VMEM((BLK * DIM,), jnp.float32),
        pltpu.VMEM((BLK * DIM,), jnp.float32),
        pltpu.VMEM((TAIL_N * DIM,), jnp.float32),
        pltpu.SemaphoreType.DMA,
        pltpu.SemaphoreType.DMA,
        pltpu.SemaphoreType.DMA,
        pltpu.SemaphoreType.DMA,
        pltpu.SemaphoreType.DMA,
        pltpu.SemaphoreType.DMA,
    ],
    compiler_params=pltpu.CompilerParams(
        use_tc_tiling_on_sc=True,
        needs_layout_passes=False,
        disable_bounds_checks=True,
    ),
)
def _retile(tt_hbm, tail_hbm, out_hbm, in0, in1, in2, ou0, ou1, ou2, tail_v,
            is0, is1, is2, os0, os1, os2):
    in_v = (in0, in1, in2)
    out_v = (ou0, ou1, ou2)
    wid = lax.axis_index("s") * NC + lax.axis_index("c")
    isem = (is0, is1, is2)
    osem = (os0, os1, os2)
    nblocks = jnp.where(wid < NFULL % NW, NFULL // NW + 1, NFULL // NW)

    iota16 = lax.iota(jnp.int32, 16)
    diag_v = [(iota16 + k) % 16 for k in range(16)]
    diag_dst = [((iota16 + k) % 16) * DIM + iota16 for k in range(16)]

    def load(i, h):
        blk = wid + i * NW
        for tr in range(DIM // 8):
            pltpu.async_copy(
                tt_hbm.at[pl.ds(tr * 8, 8), pl.ds(blk * BLK, BLK)],
                in_v[h].at[pl.ds(tr * 8, 8)],
                isem[h],
            )

    def out_slice(i):
        blk = wid + i * NW
        return out_hbm.at[pl.ds(blk * (BLK * DIM), BLK * DIM)]

    def transpose(h):
        @plsc.parallel_loop(0, BLK // 16, step=1, unroll=4)
        def _m(m):
            for dh in range(DIM // 16):
                d_vec = iota16 + (16 * dh)
                for k in range(16):
                    x = plsc.load_gather(in_v[h], [d_vec, diag_v[k] + 16 * m])
                    plsc.store_scatter(
                        out_v[h], [diag_dst[k] + (DIM * 16 * m + 16 * dh)], x
                    )

    def process(i, h, wait_out, refill):
        pltpu.make_async_copy(tt_hbm.at[:, pl.ds(0, BLK)], in_v[h], isem[h]).wait()
        if wait_out:
            pltpu.make_async_copy(out_v[h], out_slice(i), osem[h]).wait()
        transpose(h)
        pltpu.async_copy(out_v[h], out_slice(i), osem[h])
        if refill:
            @pl.when(i + 3 < nblocks)
            def _():
                load(i + 3, h)

    @pl.when(wid == 0)
    def _tail():
        pltpu.sync_copy(tail_hbm, tail_v)
        pltpu.sync_copy(tail_v, out_hbm.at[pl.ds(TAIL_V0 * DIM, TAIL_N * DIM)])

    load(0, 0)

    @pl.when(nblocks > 1)
    def _l1():
        load(1, 1)

    @pl.when(nblocks > 2)
    def _l2():
        load(2, 2)

    process(0, 0, False, True)

    @pl.when(nblocks > 1)
    def _p1():
        process(1, 1, False, True)

    @pl.when(nblocks > 2)
    def _p2():
        process(2, 2, False, True)

    @pl.loop(3, jnp.maximum(nblocks, 3), step=3)
    def _blocks(i0):
        process(i0, 0, True, True)

        @pl.when(i0 + 1 < nblocks)
        def _s1():
            process(i0 + 1, 1, True, True)

        @pl.when(i0 + 2 < nblocks)
        def _s2():
            process(i0 + 2, 2, True, True)

    pltpu.make_async_copy(out_v[0], out_slice(0), osem[0]).wait()

    @pl.when(nblocks > 1)
    def _drain1():
        pltpu.make_async_copy(out_v[1], out_slice(0), osem[1]).wait()

    @pl.when(nblocks > 2)
    def _drain2():
        pltpu.make_async_copy(out_v[2], out_slice(0), osem[2]).wait()


ROWS_PER_W = BATCH // NW
HCHUNKS = ((0, 104), (104, 96))
NGROUP = ROWS_PER_W // 4
GR = 4


@functools.partial(
    pl.kernel,
    mesh=_mesh,
    out_type=jax.ShapeDtypeStruct((BATCH, HIST, PAD), jnp.float32),
    scratch_types=[
        pltpu.VMEM((ROWS_PER_W, HIST), jnp.int32),
        pltpu.VMEM((2, GR, HIST, DIM), jnp.float32),
        pltpu.SemaphoreType.DMA,
        pltpu.SemaphoreType.DMA,
        pltpu.SemaphoreType.DMA,
        pltpu.SemaphoreType.DMA,
    ],
    compiler_params=pltpu.CompilerParams(use_tc_tiling_on_sc=False),
)
def _emb_gather(idx_hbm, table_hbm, out_hbm, idx_v, rows_v, gs0, gs1, os0, os1):
    wid = lax.axis_index("s") * NC + lax.axis_index("c")
    rbase = wid * ROWS_PER_W
    gsem = (gs0, gs1)
    osem = (os0, os1)

    pltpu.sync_copy(idx_hbm.at[pl.ds(rbase, ROWS_PER_W)], idx_v)

    def fire_gathers(g, h):
        for k in range(GR):
            for off, n in HCHUNKS:
                pltpu.async_copy(
                    table_hbm.at[idx_v.at[g * GR + k, pl.ds(off, n)]],
                    rows_v.at[h, k, pl.ds(off, n)],
                    gsem[h],
                )

    def process(g, h, refill):
        out_slice = out_hbm.at[pl.ds(rbase + g * GR, GR), :, pl.ds(0, DIM)]
        pltpu.make_async_copy(out_slice, rows_v.at[h], gsem[h]).wait()
        out_copy = pltpu.make_async_copy(rows_v.at[h], out_slice, osem[h])
        out_copy.start()
        out_copy.wait()
        if refill:
            fire_gathers(g + 2, h)

    fire_gathers(0, 0)
    fire_gathers(1, 1)

    @pl.loop(0, NGROUP - 2, step=2)
    def _groups(g0):
        process(g0, 0, True)
        process(g0 + 1, 1, True)

    process(NGROUP - 2, 0, False)
    process(NGROUP - 1, 1, False)


def kernel(token_indices, table):
    tail = table[TAIL_V0:].reshape(TAIL_N * DIM)
    tlin = _retile(table.T, tail).reshape(VOCAB, DIM)
    out = _emb_gather(token_indices.astype(jnp.int32), tlin)
    return out[:, :, :DIM]
